# baseline (speedup 1.0000x reference)
import functools

import jax
import jax.numpy as jnp
from jax import lax
from jax.experimental import pallas as pl
from jax.experimental.pallas import tpu as pltpu
from jax.experimental.pallas import tpu_sc as plsc

_BATCH = 16384
_EMBED = 64

_info = plsc.get_sparse_core_info()
_NC = _info.num_cores
_NS = _info.num_subcores
_NW = _NC * _NS
_BPW = _BATCH // _NW
_CHUNK = 128
_NCHUNK = _BPW // _CHUNK


@functools.partial(
    pl.kernel,
    mesh=plsc.VectorSubcoreMesh(core_axis_name="c", subcore_axis_name="s"),
    compiler_params=pltpu.CompilerParams(needs_layout_passes=False),
    out_type=[
        jax.ShapeDtypeStruct((_BATCH, _EMBED), jnp.float32),
        jax.ShapeDtypeStruct((_BATCH, _EMBED), jnp.float32),
        jax.ShapeDtypeStruct((_BATCH, _EMBED), jnp.float32),
    ],
    scratch_types=[
        pltpu.VMEM((_BPW,), jnp.int32),
        pltpu.SemaphoreType.DMA,
    ],
)
def _sbpr_gather(user_hbm, pos_hbm, neg_hbm, eu_hbm, ei_hbm,
                 out_u, out_p, out_n,
                 idx_v, sem_g):
    wid = lax.axis_index("s") * _NC + lax.axis_index("c")
    base = wid * _BPW

    for idx_hbm, tbl, outh in ((user_hbm, eu_hbm, out_u),
                               (pos_hbm, ei_hbm, out_p),
                               (neg_hbm, ei_hbm, out_n)):
        pltpu.sync_copy(idx_hbm.at[pl.ds(base, _BPW)], idx_v)

        for ch in range(_NCHUNK):
            def group_body(g, carry):
                j0 = ch * _CHUNK + g * 16
                v = idx_v[pl.ds(j0, 16)]
                for lane in range(16):
                    r = v[lane]
                    pltpu.async_copy(tbl.at[pl.ds(r, 1)],
                                     outh.at[pl.ds(base + j0 + lane, 1)],
                                     sem_g)
                return carry
            lax.fori_loop(0, _CHUNK // 16, group_body, 0)
            pltpu.make_async_copy(
                tbl.at[pl.ds(0, _CHUNK)],
                outh.at[pl.ds(base + ch * _CHUNK, _CHUNK)], sem_g).wait()


@jax.jit
def kernel(user, pos_item, neg_item, embed_user, embed_item):
    return tuple(_sbpr_gather(user, pos_item, neg_item,
                              embed_user, embed_item))


# speedup vs baseline: 1.9970x; 1.9970x over previous
import functools

import jax
import jax.numpy as jnp
from jax import lax
from jax.experimental import pallas as pl
from jax.experimental.pallas import tpu as pltpu
from jax.experimental.pallas import tpu_sc as plsc

_BATCH = 16384
_EMBED = 64

_info = plsc.get_sparse_core_info()
_NC = _info.num_cores
_NS = _info.num_subcores
_NW = _NC * _NS
_BPW = _BATCH // _NW
_CHUNK = 128
_NCHUNK = _BPW // _CHUNK


@functools.partial(
    pl.kernel,
    mesh=plsc.VectorSubcoreMesh(core_axis_name="c", subcore_axis_name="s"),
    compiler_params=pltpu.CompilerParams(needs_layout_passes=False),
    out_type=[
        jax.ShapeDtypeStruct((_BATCH, _EMBED), jnp.float32),
        jax.ShapeDtypeStruct((_BATCH, _EMBED), jnp.float32),
        jax.ShapeDtypeStruct((_BATCH, _EMBED), jnp.float32),
    ],
    scratch_types=[
        pltpu.VMEM((_BPW,), jnp.int32),
        pltpu.VMEM((_BPW,), jnp.int32),
        pltpu.VMEM((_BPW,), jnp.int32),
        pltpu.VMEM((_BPW, _EMBED), jnp.float32),
        pltpu.SemaphoreType.DMA,
        pltpu.SemaphoreType.DMA,
    ],
)
def _sbpr_gather(user_hbm, pos_hbm, neg_hbm, eu_hbm, ei_hbm,
                 out_u, out_p, out_n,
                 idx_u, idx_p, idx_n, rows_v, sem_g, sem_o):
    wid = lax.axis_index("s") * _NC + lax.axis_index("c")
    base = wid * _BPW

    pltpu.sync_copy(user_hbm.at[pl.ds(base, _BPW)], idx_u)
    pltpu.sync_copy(pos_hbm.at[pl.ds(base, _BPW)], idx_p)
    pltpu.sync_copy(neg_hbm.at[pl.ds(base, _BPW)], idx_n)

    prev_out = None
    for idx_v, tbl, outh in ((idx_u, eu_hbm, out_u),
                             (idx_p, ei_hbm, out_p),
                             (idx_n, ei_hbm, out_n)):
        if prev_out is not None:
            prev_out.wait()
            prev_out = None
        for ch in range(_NCHUNK):
            def group_body(g, carry):
                j0 = ch * _CHUNK + g * 16
                v = idx_v[pl.ds(j0, 16)]
                for lane in range(16):
                    r = v[lane]
                    pltpu.async_copy(tbl.at[pl.ds(r, 1)],
                                     rows_v.at[pl.ds(j0 + lane, 1)],
                                     sem_g)
                return carry
            lax.fori_loop(0, _CHUNK // 16, group_body, 0)
            pltpu.make_async_copy(
                tbl.at[pl.ds(0, _CHUNK)],
                rows_v.at[pl.ds(ch * _CHUNK, _CHUNK)], sem_g).wait()
        prev_out = pltpu.async_copy(rows_v, outh.at[pl.ds(base, _BPW)],
                                    sem_o)
    prev_out.wait()


@jax.jit
def kernel(user, pos_item, neg_item, embed_user, embed_item):
    return tuple(_sbpr_gather(user, pos_item, neg_item,
                              embed_user, embed_item))
